# baseline (speedup 1.0000x reference)
import jax
import jax.numpy as jnp
from jax import lax
from jax.experimental import pallas as pl
from jax.experimental.pallas import tpu as pltpu
from jax.experimental.pallas import tpu_sc as plsc

_H = 128
_NE = 8192
_D = 32
_BETA = 0.25
_NTOK = 6272

_RB = 448
_NB = _NTOK // _RB
_CT = 2048
_NT = _NE // _CT
_HI = 128
_LO = 64

_NC, _NS, _L = 2, 16, 16
_NW = _NC * _NS
_P = 208
_BPAD = _P * _NW
_PC = 104
_CW = 128


def _conv(x, w, b, stride, pad):
    wt = jnp.transpose(w, (2, 3, 1, 0))
    out = lax.conv_general_dilated(
        x, wt, (stride, stride), [(pad, pad), (pad, pad)],
        dimension_numbers=('NHWC', 'HWIO', 'NHWC'))
    return out + b[None, None, None, :]


def _conv_t(x, w, b):
    O, I = w.shape[0], w.shape[1]
    wt = jnp.transpose(w, (2, 3, 1, 0))
    w6 = wt.reshape(2, 2, 2, 2, I, O)
    w2 = jnp.transpose(w6, (0, 2, 4, 1, 3, 5)).reshape(2, 2, I, 4 * O)
    y = lax.conv_general_dilated(
        x, w2, (1, 1), [(1, 1), (1, 1)],
        dimension_numbers=('NHWC', 'HWIO', 'NHWC'))
    B, H1, W1, _ = y.shape
    H, W = H1 - 1, W1 - 1
    parts = []
    for a in (0, 1):
        row = []
        for bb in (0, 1):
            c0 = (2 * a + bb) * O
            row.append(y[:, a:a + H, bb:bb + W, c0:c0 + O])
        parts.append(jnp.stack(row, axis=3))
    y4 = jnp.stack(parts, axis=3)
    out = jnp.transpose(y4, (0, 1, 3, 2, 4, 5)).reshape(B, 2 * H, 2 * W, O)
    return out + b[None, None, None, :]


def _dist_body(z_ref, cb_ref, z2_ref, c2_ref, idx_ref, loss_ref, perp_ref,
               cnt_ref):
    z = z_ref[...]
    z2 = z2_ref[...]
    run_min = jnp.full((_RB,), jnp.inf, jnp.float32)
    run_arg = jnp.zeros((_RB,), jnp.int32)
    for t in range(_NT):
        cb_t = cb_ref[pl.ds(t * _CT, _CT), :]
        c2_t = c2_ref[0, pl.ds(t * _CT, _CT)]
        s = lax.dot_general(z, cb_t, (((1,), (1,)), ((), ())),
                            preferred_element_type=jnp.float32)
        d = (z2 + c2_t[None, :]) - 2.0 * s
        m = jnp.min(d, axis=1)
        cols = lax.broadcasted_iota(jnp.int32, d.shape, 1)
        a = jnp.min(jnp.where(d == m[:, None], cols, _CT), axis=1) + t * _CT
        upd = m < run_min
        run_arg = jnp.where(upd, a, run_arg)
        run_min = jnp.where(upd, m, run_min)
    idx_ref[0, 0, :] = run_arg

    @pl.when(pl.program_id(0) == 0)
    def _init():
        loss_ref[...] = jnp.zeros((1, 1), jnp.float32)
        cnt_ref[...] = jnp.zeros((_HI, _LO), jnp.float32)
    loss_ref[...] += jnp.sum(run_min).reshape(1, 1)
    hi = run_arg[:, None] >> 6
    lo = run_arg[:, None] & 63
    hit_hi = (hi == lax.broadcasted_iota(jnp.int32, (1, _HI), 1)
              ).astype(jnp.float32)
    hit_lo = (lo == lax.broadcasted_iota(jnp.int32, (1, _LO), 1)
              ).astype(jnp.float32)
    cnt_ref[...] += lax.dot_general(
        hit_hi, hit_lo, (((0,), (0,)), ((), ())),
        preferred_element_type=jnp.float32)

    @pl.when(pl.program_id(0) == _NB - 1)
    def _fin():
        e_mean = cnt_ref[...] / _NTOK
        ent_sum = jnp.sum(e_mean * jnp.log(e_mean + 1e-10))
        perp_ref[...] = jnp.exp(-ent_sum).reshape(1, 1)
        loss_ref[...] = loss_ref[...] * ((1.0 + _BETA) / (_NTOK * _D))


def _vq_argmin(zf, codebook, z2, c2):
    return pl.pallas_call(
        _dist_body,
        grid=(_NB,),
        in_specs=[
            pl.BlockSpec((_RB, _D), lambda i: (i, 0)),
            pl.BlockSpec((_NE, _D), lambda i: (0, 0)),
            pl.BlockSpec((_RB, 1), lambda i: (i, 0)),
            pl.BlockSpec((1, _NE), lambda i: (0, 0)),
        ],
        out_specs=[
            pl.BlockSpec((1, 1, _RB), lambda i: (i, 0, 0)),
            pl.BlockSpec((1, 1), lambda i: (0, 0)),
            pl.BlockSpec((1, 1), lambda i: (0, 0)),
        ],
        out_shape=[
            jax.ShapeDtypeStruct((_NB, 1, _RB), jnp.int32),
            jax.ShapeDtypeStruct((1, 1), jnp.float32),
            jax.ShapeDtypeStruct((1, 1), jnp.float32),
        ],
        scratch_shapes=[pltpu.VMEM((_HI, _LO), jnp.float32)],
    )(zf, codebook, z2, c2)


def _sc_body(cb_hbm, idx_hbm, zq_hbm, idx_v, rows_v, sem):
    cid = lax.axis_index("c")
    sid = lax.axis_index("s")
    wid = sid * _NC + cid
    base = wid * _P
    for j in range(_P // _PC):
        pltpu.sync_copy(idx_hbm.at[pl.ds(base + j * _PC, _PC)], idx_v.at[j])
    for j in range(_P // _PC):
        pltpu.async_copy(cb_hbm.at[idx_v.at[j]],
                         rows_v.at[pl.ds(j * _PC, _PC)], sem).wait()
    pltpu.sync_copy(rows_v, zq_hbm.at[pl.ds(base, _P)])


def _sc_quantize(codebook, idx_pad):
    mesh = plsc.VectorSubcoreMesh(core_axis_name="c", subcore_axis_name="s")
    kern = pl.kernel(
        _sc_body,
        out_type=jax.ShapeDtypeStruct((_BPAD, _CW), jnp.float32),
        mesh=mesh,
        scratch_types=[
            pltpu.VMEM((_P // _PC, _PC), jnp.int32),
            pltpu.VMEM((_P, _CW), jnp.float32),
            pltpu.SemaphoreType.DMA,
        ],
    )
    return kern(codebook, idx_pad)


def kernel(x, enc_w1, enc_b1, enc_w2, enc_b2, enc_w3, enc_b3, pre_w, pre_b,
           codebook, dec_w1, dec_b1, dec_w2, dec_b2, dec_w3, dec_b3):
    xh = jnp.transpose(x, (0, 2, 3, 1))
    z = jax.nn.relu(_conv(xh, enc_w1, enc_b1, 2, 1))
    z = jax.nn.relu(_conv(z, enc_w2, enc_b2, 2, 1))
    z = _conv(z, enc_w3, enc_b3, 1, 1)
    z_e = _conv(z, pre_w, pre_b, 1, 0)
    B, Hh, Ww, C = z_e.shape
    z_flat = z_e.reshape(-1, C)

    z2 = jnp.sum(z_flat ** 2, axis=1, keepdims=True)
    c2 = jnp.sum(codebook ** 2, axis=1)[None, :]

    idx3, emb_loss, perp = _vq_argmin(z_flat, codebook, z2, c2)
    idx = idx3.reshape(-1)
    idx_pad = jnp.concatenate(
        [idx, jnp.zeros((_BPAD - _NTOK,), jnp.int32)])

    cb_pad = jnp.concatenate(
        [codebook, jnp.zeros((_NE, _CW - _D), jnp.float32)], axis=1)
    zq_pad = _sc_quantize(cb_pad, idx_pad)
    z_q_flat = zq_pad[:_NTOK, :_D]
    z_q = z_q_flat.reshape(B, Hh, Ww, C)

    z_q_st = z_e + lax.stop_gradient(z_q - z_e)
    h = _conv(z_q_st, dec_w1, dec_b1, 1, 1)
    h = jax.nn.relu(_conv_t(h, dec_w2, dec_b2))
    x_hat = _conv_t(h, dec_w3, dec_b3)
    x_hat = jnp.transpose(x_hat, (0, 3, 1, 2))
    return emb_loss[0, 0], x_hat, perp[0, 0]


# speedup vs baseline: 1.3031x; 1.3031x over previous
import jax
import jax.numpy as jnp
from jax import lax
from jax.experimental import pallas as pl
from jax.experimental.pallas import tpu as pltpu
from jax.experimental.pallas import tpu_sc as plsc

_H = 128
_NE = 8192
_D = 32
_BETA = 0.25
_NTOK = 6272

_RB = 448
_NH = _NTOK // 2
_NB = _NH // _RB
_CT = 2048
_NT = _NE // _CT
_HI = 128
_LO = 64

_NC, _NS, _L = 2, 16, 16
_NW = _NC * _NS
_P = 104
_BPAD = _P * _NW
_PC = 104
_CW = 128


def _conv(x, w, b, stride, pad):
    wt = jnp.transpose(w, (2, 3, 1, 0))
    out = lax.conv_general_dilated(
        x, wt, (stride, stride), [(pad, pad), (pad, pad)],
        dimension_numbers=('NHWC', 'HWIO', 'NHWC'))
    return out + b[None, None, None, :]


def _conv_t(x, w, b, stride, pad):
    k = w.shape[2]
    p = k - 1 - pad
    wt = jnp.transpose(w, (2, 3, 1, 0))
    out = lax.conv_general_dilated(
        x, wt, (1, 1), [(p, p), (p, p)], lhs_dilation=(stride, stride),
        dimension_numbers=('NHWC', 'HWIO', 'NHWC'))
    return out + b[None, None, None, :]


def _argmin_block(z_ref, cb_ref, z2_ref, c2_ref):
    z = z_ref[...]
    z2 = z2_ref[...]
    run_min = jnp.full((_RB,), jnp.inf, jnp.float32)
    run_arg = jnp.zeros((_RB,), jnp.int32)
    for t in range(_NT):
        cb_t = cb_ref[pl.ds(t * _CT, _CT), :]
        c2_t = c2_ref[0, pl.ds(t * _CT, _CT)]
        s = lax.dot_general(z, cb_t, (((1,), (1,)), ((), ())),
                            preferred_element_type=jnp.float32)
        d = (z2 + c2_t[None, :]) - 2.0 * s
        m = jnp.min(d, axis=1)
        cols = lax.broadcasted_iota(jnp.int32, d.shape, 1)
        a = jnp.min(jnp.where(d == m[:, None], cols, _CT), axis=1) + t * _CT
        upd = m < run_min
        run_arg = jnp.where(upd, a, run_arg)
        run_min = jnp.where(upd, m, run_min)
    return run_min, run_arg


def _hist_update(run_arg):
    hi = run_arg[:, None] >> 6
    lo = run_arg[:, None] & 63
    hit_hi = (hi == lax.broadcasted_iota(jnp.int32, (1, _HI), 1)
              ).astype(jnp.float32)
    hit_lo = (lo == lax.broadcasted_iota(jnp.int32, (1, _LO), 1)
              ).astype(jnp.float32)
    return lax.dot_general(hit_hi, hit_lo, (((0,), (0,)), ((), ())),
                           preferred_element_type=jnp.float32)


def _dist_body_a(z_ref, cb_ref, z2_ref, c2_ref, idx_ref, loss_ref, cnt_ref):
    run_min, run_arg = _argmin_block(z_ref, cb_ref, z2_ref, c2_ref)
    idx_ref[0, 0, :] = run_arg

    @pl.when(pl.program_id(0) == 0)
    def _init():
        loss_ref[...] = jnp.zeros((1, 1), jnp.float32)
        cnt_ref[...] = jnp.zeros((_HI, _LO), jnp.float32)
    loss_ref[...] += jnp.sum(run_min).reshape(1, 1)
    cnt_ref[...] += _hist_update(run_arg)


def _dist_body_b(z_ref, cb_ref, z2_ref, c2_ref, lin_ref, cin_ref,
                 idx_ref, loss_ref, perp_ref, cnt_ref):
    run_min, run_arg = _argmin_block(z_ref, cb_ref, z2_ref, c2_ref)
    idx_ref[0, 0, :] = run_arg

    @pl.when(pl.program_id(0) == 0)
    def _init():
        loss_ref[...] = lin_ref[...]
        cnt_ref[...] = cin_ref[...]
    loss_ref[...] += jnp.sum(run_min).reshape(1, 1)
    cnt_ref[...] += _hist_update(run_arg)

    @pl.when(pl.program_id(0) == _NB - 1)
    def _fin():
        e_mean = cnt_ref[...] / _NTOK
        ent_sum = jnp.sum(e_mean * jnp.log(e_mean + 1e-10))
        perp_ref[...] = jnp.exp(-ent_sum).reshape(1, 1)
        loss_ref[...] = loss_ref[...] * ((1.0 + _BETA) / (_NTOK * _D))


_ZSPEC = [
    pl.BlockSpec((_RB, _D), lambda i: (i, 0)),
    pl.BlockSpec((_NE, _D), lambda i: (0, 0)),
    pl.BlockSpec((_RB, 1), lambda i: (i, 0)),
    pl.BlockSpec((1, _NE), lambda i: (0, 0)),
]
_SSPEC = pl.BlockSpec((1, 1), lambda i: (0, 0))
_CSPEC = pl.BlockSpec((_HI, _LO), lambda i: (0, 0))
_ISPEC = pl.BlockSpec((1, 1, _RB), lambda i: (i, 0, 0))


def _vq_argmin_a(zf, codebook, z2, c2):
    return pl.pallas_call(
        _dist_body_a,
        grid=(_NB,),
        in_specs=_ZSPEC,
        out_specs=[_ISPEC, _SSPEC, _CSPEC],
        out_shape=[
            jax.ShapeDtypeStruct((_NB, 1, _RB), jnp.int32),
            jax.ShapeDtypeStruct((1, 1), jnp.float32),
            jax.ShapeDtypeStruct((_HI, _LO), jnp.float32),
        ],
    )(zf, codebook, z2, c2)


def _vq_argmin_b(zf, codebook, z2, c2, loss_in, cnt_in):
    return pl.pallas_call(
        _dist_body_b,
        grid=(_NB,),
        in_specs=_ZSPEC + [_SSPEC, _CSPEC],
        out_specs=[_ISPEC, _SSPEC, _SSPEC],
        out_shape=[
            jax.ShapeDtypeStruct((_NB, 1, _RB), jnp.int32),
            jax.ShapeDtypeStruct((1, 1), jnp.float32),
            jax.ShapeDtypeStruct((1, 1), jnp.float32),
        ],
        scratch_shapes=[pltpu.VMEM((_HI, _LO), jnp.float32)],
    )(zf, codebook, z2, c2, loss_in, cnt_in)


def _sc_body(cb_hbm, idx_hbm, zq_hbm, idx_v, rows_v, sem):
    cid = lax.axis_index("c")
    sid = lax.axis_index("s")
    wid = sid * _NC + cid
    base = wid * _P
    for j in range(_P // _PC):
        pltpu.sync_copy(idx_hbm.at[pl.ds(base + j * _PC, _PC)], idx_v.at[j])
    for j in range(_P // _PC):
        pltpu.async_copy(cb_hbm.at[idx_v.at[j]],
                         rows_v.at[pl.ds(j * _PC, _PC)], sem).wait()
    pltpu.sync_copy(rows_v, zq_hbm.at[pl.ds(base, _P)])


def _sc_quantize(codebook, idx_pad):
    mesh = plsc.VectorSubcoreMesh(core_axis_name="c", subcore_axis_name="s")
    kern = pl.kernel(
        _sc_body,
        out_type=jax.ShapeDtypeStruct((_BPAD, _CW), jnp.float32),
        mesh=mesh,
        scratch_types=[
            pltpu.VMEM((_P // _PC, _PC), jnp.int32),
            pltpu.VMEM((_P, _CW), jnp.float32),
            pltpu.SemaphoreType.DMA,
        ],
    )
    return kern(codebook, idx_pad)


def kernel(x, enc_w1, enc_b1, enc_w2, enc_b2, enc_w3, enc_b3, pre_w, pre_b,
           codebook, dec_w1, dec_b1, dec_w2, dec_b2, dec_w3, dec_b3):
    xh = jnp.transpose(x, (0, 2, 3, 1))
    z = jax.nn.relu(_conv(xh, enc_w1, enc_b1, 2, 1))
    z = jax.nn.relu(_conv(z, enc_w2, enc_b2, 2, 1))
    z = _conv(z, enc_w3, enc_b3, 1, 1)
    z_e = _conv(z, pre_w, pre_b, 1, 0)
    B, Hh, Ww, C = z_e.shape
    z_flat = z_e.reshape(-1, C)

    z2 = jnp.sum(z_flat ** 2, axis=1, keepdims=True)
    c2 = jnp.sum(codebook ** 2, axis=1)[None, :]
    cb_pad = jnp.concatenate(
        [codebook, jnp.zeros((_NE, _CW - _D), jnp.float32)], axis=1)

    idx3a, loss_a, cnt_a = _vq_argmin_a(
        z_flat[:_NH], codebook, z2[:_NH], c2)
    idxa_pad = jnp.concatenate(
        [idx3a.reshape(-1), jnp.zeros((_BPAD - _NH,), jnp.int32)])
    zqa = _sc_quantize(cb_pad, idxa_pad)

    idx3b, emb_loss, perp = _vq_argmin_b(
        z_flat[_NH:], codebook, z2[_NH:], c2, loss_a, cnt_a)
    idxb_pad = jnp.concatenate(
        [idx3b.reshape(-1), jnp.zeros((_BPAD - _NH,), jnp.int32)])
    zqb = _sc_quantize(cb_pad, idxb_pad)

    z_q_flat = jnp.concatenate([zqa[:_NH, :_D], zqb[:_NH, :_D]])
    z_q = z_q_flat.reshape(B, Hh, Ww, C)

    z_q_st = z_e + lax.stop_gradient(z_q - z_e)
    h = _conv(z_q_st, dec_w1, dec_b1, 1, 1)
    h = jax.nn.relu(_conv_t(h, dec_w2, dec_b2, 2, 1))
    x_hat = _conv_t(h, dec_w3, dec_b3, 2, 1)
    x_hat = jnp.transpose(x_hat, (0, 3, 1, 2))
    return emb_loss[0, 0], x_hat, perp[0, 0]


# speedup vs baseline: 1.3945x; 1.0702x over previous
import jax
import jax.numpy as jnp
from jax import lax
from jax.experimental import pallas as pl
from jax.experimental.pallas import tpu as pltpu
from jax.experimental.pallas import tpu_sc as plsc

_H = 128
_NE = 8192
_D = 32
_BETA = 0.25
_NTOK = 6272

_RB = 448
_NB = _NTOK // _RB
_CT = 2048
_NT = _NE // _CT
_HI = 128
_LO = 64

_NC, _NS, _L = 2, 16, 16
_NW = _NC * _NS
_P = 208
_BPAD = _P * _NW
_PC = 104
_CW = 128


def _conv(x, w, b, stride, pad):
    wt = jnp.transpose(w, (2, 3, 1, 0))
    out = lax.conv_general_dilated(
        x, wt, (stride, stride), [(pad, pad), (pad, pad)],
        dimension_numbers=('NHWC', 'HWIO', 'NHWC'))
    return out + b[None, None, None, :]


def _conv_t(x, w, b, stride, pad):
    k = w.shape[2]
    p = k - 1 - pad
    wt = jnp.transpose(w, (2, 3, 1, 0))
    out = lax.conv_general_dilated(
        x, wt, (1, 1), [(p, p), (p, p)], lhs_dilation=(stride, stride),
        dimension_numbers=('NHWC', 'HWIO', 'NHWC'))
    return out + b[None, None, None, :]


def _argmin_block(z_ref, cb_ref, z2_ref, c2_ref):
    z = z_ref[...]
    z2 = z2_ref[...]
    run_min = jnp.full((_RB,), jnp.inf, jnp.float32)
    run_arg = jnp.zeros((_RB,), jnp.int32)
    for t in range(_NT):
        cb_t = cb_ref[pl.ds(t * _CT, _CT), :]
        c2_t = c2_ref[0, pl.ds(t * _CT, _CT)]
        s = lax.dot_general(z, cb_t, (((1,), (1,)), ((), ())),
                            preferred_element_type=jnp.float32)
        d = (z2 + c2_t[None, :]) - 2.0 * s
        m = jnp.min(d, axis=1)
        cols = lax.broadcasted_iota(jnp.int32, d.shape, 1)
        a = jnp.min(jnp.where(d == m[:, None], cols, _CT), axis=1) + t * _CT
        upd = m < run_min
        run_arg = jnp.where(upd, a, run_arg)
        run_min = jnp.where(upd, m, run_min)
    return run_min, run_arg


def _hist_update(run_arg):
    hi = run_arg[:, None] >> 6
    lo = run_arg[:, None] & 63
    hit_hi = (hi == lax.broadcasted_iota(jnp.int32, (1, _HI), 1)
              ).astype(jnp.float32)
    hit_lo = (lo == lax.broadcasted_iota(jnp.int32, (1, _LO), 1)
              ).astype(jnp.float32)
    return lax.dot_general(hit_hi, hit_lo, (((0,), (0,)), ((), ())),
                           preferred_element_type=jnp.float32)


def _dist_body(z_ref, cb_ref, z2_ref, c2_ref, idx_ref, loss_ref, perp_ref,
               cnt_ref):
    run_min, run_arg = _argmin_block(z_ref, cb_ref, z2_ref, c2_ref)
    idx_ref[0, 0, :] = run_arg

    @pl.when(pl.program_id(0) == 0)
    def _init():
        loss_ref[...] = jnp.zeros((1, 1), jnp.float32)
        cnt_ref[...] = jnp.zeros((_HI, _LO), jnp.float32)
    loss_ref[...] += jnp.sum(run_min).reshape(1, 1)
    cnt_ref[...] += _hist_update(run_arg)

    @pl.when(pl.program_id(0) == _NB - 1)
    def _fin():
        e_mean = cnt_ref[...] / _NTOK
        ent_sum = jnp.sum(e_mean * jnp.log(e_mean + 1e-10))
        perp_ref[...] = jnp.exp(-ent_sum).reshape(1, 1)
        loss_ref[...] = loss_ref[...] * ((1.0 + _BETA) / (_NTOK * _D))


def _vq_argmin(zf, codebook, z2, c2):
    return pl.pallas_call(
        _dist_body,
        grid=(_NB,),
        in_specs=[
            pl.BlockSpec((_RB, _D), lambda i: (i, 0)),
            pl.BlockSpec((_NE, _D), lambda i: (0, 0)),
            pl.BlockSpec((_RB, 1), lambda i: (i, 0)),
            pl.BlockSpec((1, _NE), lambda i: (0, 0)),
        ],
        out_specs=[
            pl.BlockSpec((1, 1, _RB), lambda i: (i, 0, 0)),
            pl.BlockSpec((1, 1), lambda i: (0, 0)),
            pl.BlockSpec((1, 1), lambda i: (0, 0)),
        ],
        out_shape=[
            jax.ShapeDtypeStruct((_NB, 1, _RB), jnp.int32),
            jax.ShapeDtypeStruct((1, 1), jnp.float32),
            jax.ShapeDtypeStruct((1, 1), jnp.float32),
        ],
        scratch_shapes=[pltpu.VMEM((_HI, _LO), jnp.float32)],
    )(zf, codebook, z2, c2)


def _sc_body(cb_hbm, idx_hbm, zq_hbm, idx_v, rows_v, sem0, sem1):
    cid = lax.axis_index("c")
    sid = lax.axis_index("s")
    wid = sid * _NC + cid
    base = wid * _P
    for j in range(_P // _PC):
        pltpu.sync_copy(idx_hbm.at[pl.ds(base + j * _PC, _PC)], idx_v.at[j])
    sems = (sem0, sem1)
    cps = [pltpu.async_copy(cb_hbm.at[idx_v.at[j]],
                            rows_v.at[pl.ds(j * _PC, _PC)], sems[j])
           for j in range(_P // _PC)]
    for cp in cps:
        cp.wait()
    pltpu.sync_copy(rows_v, zq_hbm.at[pl.ds(base, _P)])


def _sc_quantize(codebook, idx_pad):
    mesh = plsc.VectorSubcoreMesh(core_axis_name="c", subcore_axis_name="s")
    kern = pl.kernel(
        _sc_body,
        out_type=jax.ShapeDtypeStruct((_BPAD, _CW), jnp.float32),
        mesh=mesh,
        scratch_types=[
            pltpu.VMEM((_P // _PC, _PC), jnp.int32),
            pltpu.VMEM((_P, _CW), jnp.float32),
            pltpu.SemaphoreType.DMA,
            pltpu.SemaphoreType.DMA,
        ],
    )
    return kern(codebook, idx_pad)


def kernel(x, enc_w1, enc_b1, enc_w2, enc_b2, enc_w3, enc_b3, pre_w, pre_b,
           codebook, dec_w1, dec_b1, dec_w2, dec_b2, dec_w3, dec_b3):
    xh = jnp.transpose(x, (0, 2, 3, 1))
    z = jax.nn.relu(_conv(xh, enc_w1, enc_b1, 2, 1))
    z = jax.nn.relu(_conv(z, enc_w2, enc_b2, 2, 1))
    z = _conv(z, enc_w3, enc_b3, 1, 1)
    z_e = _conv(z, pre_w, pre_b, 1, 0)
    B, Hh, Ww, C = z_e.shape
    z_flat = z_e.reshape(-1, C)

    z2 = jnp.sum(z_flat ** 2, axis=1, keepdims=True)
    c2 = jnp.sum(codebook ** 2, axis=1)[None, :]

    idx3, emb_loss, perp = _vq_argmin(z_flat, codebook, z2, c2)
    idx = idx3.reshape(-1)
    idx_pad = jnp.concatenate(
        [idx, jnp.zeros((_BPAD - _NTOK,), jnp.int32)])

    cb_pad = jnp.concatenate(
        [codebook, jnp.zeros((_NE, _CW - _D), jnp.float32)], axis=1)
    zq_pad = _sc_quantize(cb_pad, idx_pad)
    z_q_flat = zq_pad[:_NTOK, :_D]
    z_q = z_q_flat.reshape(B, Hh, Ww, C)

    z_q_st = z_e + lax.stop_gradient(z_q - z_e)
    h = _conv(z_q_st, dec_w1, dec_b1, 1, 1)
    h = jax.nn.relu(_conv_t(h, dec_w2, dec_b2, 2, 1))
    x_hat = _conv_t(h, dec_w3, dec_b3, 2, 1)
    x_hat = jnp.transpose(x_hat, (0, 3, 1, 2))
    return emb_loss[0, 0], x_hat, perp[0, 0]


# speedup vs baseline: 1.4064x; 1.0085x over previous
import jax
import jax.numpy as jnp
from jax import lax
from jax.experimental import pallas as pl
from jax.experimental.pallas import tpu as pltpu
from jax.experimental.pallas import tpu_sc as plsc

_H = 128
_NE = 8192
_D = 32
_BETA = 0.25
_NTOK = 6272

_RB = 896
_NB = _NTOK // _RB
_CT = 2048
_NT = _NE // _CT
_HI = 128
_LO = 64

_NC, _NS, _L = 2, 16, 16
_NW = _NC * _NS
_P = 208
_BPAD = _P * _NW
_PC = 104
_CW = 128


def _conv(x, w, b, stride, pad):
    wt = jnp.transpose(w, (2, 3, 1, 0))
    out = lax.conv_general_dilated(
        x, wt, (stride, stride), [(pad, pad), (pad, pad)],
        dimension_numbers=('NHWC', 'HWIO', 'NHWC'))
    return out + b[None, None, None, :]


def _conv_t(x, w, b, stride, pad):
    k = w.shape[2]
    p = k - 1 - pad
    wt = jnp.transpose(w, (2, 3, 1, 0))
    out = lax.conv_general_dilated(
        x, wt, (1, 1), [(p, p), (p, p)], lhs_dilation=(stride, stride),
        dimension_numbers=('NHWC', 'HWIO', 'NHWC'))
    return out + b[None, None, None, :]


def _argmin_block(z_ref, cb_ref, z2_ref, c2_ref):
    z = z_ref[...]
    z2 = z2_ref[...]
    run_min = jnp.full((_RB,), jnp.inf, jnp.float32)
    run_arg = jnp.zeros((_RB,), jnp.int32)
    for t in range(_NT):
        cb_t = cb_ref[pl.ds(t * _CT, _CT), :]
        c2_t = c2_ref[0, pl.ds(t * _CT, _CT)]
        s = lax.dot_general(z, cb_t, (((1,), (1,)), ((), ())),
                            preferred_element_type=jnp.float32)
        d = (z2 + c2_t[None, :]) - 2.0 * s
        m = jnp.min(d, axis=1)
        cols = lax.broadcasted_iota(jnp.int32, d.shape, 1)
        a = jnp.min(jnp.where(d == m[:, None], cols, _CT), axis=1) + t * _CT
        upd = m < run_min
        run_arg = jnp.where(upd, a, run_arg)
        run_min = jnp.where(upd, m, run_min)
    return run_min, run_arg


def _hist_update(run_arg):
    hi = run_arg[:, None] >> 6
    lo = run_arg[:, None] & 63
    hit_hi = (hi == lax.broadcasted_iota(jnp.int32, (1, _HI), 1)
              ).astype(jnp.float32)
    hit_lo = (lo == lax.broadcasted_iota(jnp.int32, (1, _LO), 1)
              ).astype(jnp.float32)
    return lax.dot_general(hit_hi, hit_lo, (((0,), (0,)), ((), ())),
                           preferred_element_type=jnp.float32)


def _dist_body(z_ref, cb_ref, z2_ref, c2_ref, idx_ref, loss_ref, perp_ref,
               cnt_ref):
    run_min, run_arg = _argmin_block(z_ref, cb_ref, z2_ref, c2_ref)
    idx_ref[0, 0, :] = run_arg

    @pl.when(pl.program_id(0) == 0)
    def _init():
        loss_ref[...] = jnp.zeros((1, 1), jnp.float32)
        cnt_ref[...] = jnp.zeros((_HI, _LO), jnp.float32)
    loss_ref[...] += jnp.sum(run_min).reshape(1, 1)
    cnt_ref[...] += _hist_update(run_arg)

    @pl.when(pl.program_id(0) == _NB - 1)
    def _fin():
        e_mean = cnt_ref[...] / _NTOK
        ent_sum = jnp.sum(e_mean * jnp.log(e_mean + 1e-10))
        perp_ref[...] = jnp.exp(-ent_sum).reshape(1, 1)
        loss_ref[...] = loss_ref[...] * ((1.0 + _BETA) / (_NTOK * _D))


def _vq_argmin(zf, codebook, z2, c2):
    return pl.pallas_call(
        _dist_body,
        grid=(_NB,),
        in_specs=[
            pl.BlockSpec((_RB, _D), lambda i: (i, 0)),
            pl.BlockSpec((_NE, _D), lambda i: (0, 0)),
            pl.BlockSpec((_RB, 1), lambda i: (i, 0)),
            pl.BlockSpec((1, _NE), lambda i: (0, 0)),
        ],
        out_specs=[
            pl.BlockSpec((1, 1, _RB), lambda i: (i, 0, 0)),
            pl.BlockSpec((1, 1), lambda i: (0, 0)),
            pl.BlockSpec((1, 1), lambda i: (0, 0)),
        ],
        out_shape=[
            jax.ShapeDtypeStruct((_NB, 1, _RB), jnp.int32),
            jax.ShapeDtypeStruct((1, 1), jnp.float32),
            jax.ShapeDtypeStruct((1, 1), jnp.float32),
        ],
        scratch_shapes=[pltpu.VMEM((_HI, _LO), jnp.float32)],
    )(zf, codebook, z2, c2)


def _sc_body(cb_hbm, idx_hbm, zq_hbm, idx_v, rows_v, sem0, sem1):
    cid = lax.axis_index("c")
    sid = lax.axis_index("s")
    wid = sid * _NC + cid
    base = wid * _P
    for j in range(_P // _PC):
        pltpu.sync_copy(idx_hbm.at[pl.ds(base + j * _PC, _PC)], idx_v.at[j])
    sems = (sem0, sem1)
    cps = [pltpu.async_copy(cb_hbm.at[idx_v.at[j]],
                            rows_v.at[pl.ds(j * _PC, _PC)], sems[j])
           for j in range(_P // _PC)]
    for cp in cps:
        cp.wait()
    pltpu.sync_copy(rows_v, zq_hbm.at[pl.ds(base, _P)])


def _sc_quantize(codebook, idx_pad):
    mesh = plsc.VectorSubcoreMesh(core_axis_name="c", subcore_axis_name="s")
    kern = pl.kernel(
        _sc_body,
        out_type=jax.ShapeDtypeStruct((_BPAD, _CW), jnp.float32),
        mesh=mesh,
        scratch_types=[
            pltpu.VMEM((_P // _PC, _PC), jnp.int32),
            pltpu.VMEM((_P, _CW), jnp.float32),
            pltpu.SemaphoreType.DMA,
            pltpu.SemaphoreType.DMA,
        ],
    )
    return kern(codebook, idx_pad)


def kernel(x, enc_w1, enc_b1, enc_w2, enc_b2, enc_w3, enc_b3, pre_w, pre_b,
           codebook, dec_w1, dec_b1, dec_w2, dec_b2, dec_w3, dec_b3):
    xh = jnp.transpose(x, (0, 2, 3, 1))
    z = jax.nn.relu(_conv(xh, enc_w1, enc_b1, 2, 1))
    z = jax.nn.relu(_conv(z, enc_w2, enc_b2, 2, 1))
    z = _conv(z, enc_w3, enc_b3, 1, 1)
    z_e = _conv(z, pre_w, pre_b, 1, 0)
    B, Hh, Ww, C = z_e.shape
    z_flat = z_e.reshape(-1, C)

    z2 = jnp.sum(z_flat ** 2, axis=1, keepdims=True)
    c2 = jnp.sum(codebook ** 2, axis=1)[None, :]

    idx3, emb_loss, perp = _vq_argmin(z_flat, codebook, z2, c2)
    idx = idx3.reshape(-1)
    idx_pad = jnp.concatenate(
        [idx, jnp.zeros((_BPAD - _NTOK,), jnp.int32)])

    cb_pad = jnp.concatenate(
        [codebook, jnp.zeros((_NE, _CW - _D), jnp.float32)], axis=1)
    zq_pad = _sc_quantize(cb_pad, idx_pad)
    z_q_flat = zq_pad[:_NTOK, :_D]
    z_q = z_q_flat.reshape(B, Hh, Ww, C)

    z_q_st = z_e + lax.stop_gradient(z_q - z_e)
    h = _conv(z_q_st, dec_w1, dec_b1, 1, 1)
    h = jax.nn.relu(_conv_t(h, dec_w2, dec_b2, 2, 1))
    x_hat = _conv_t(h, dec_w3, dec_b3, 2, 1)
    x_hat = jnp.transpose(x_hat, (0, 3, 1, 2))
    return emb_loss[0, 0], x_hat, perp[0, 0]
